# Initial kernel scaffold; baseline (speedup 1.0000x reference)
import jax
import jax.numpy as jnp
from jax.experimental import pallas as pl


def kernel(x, edge_index, W_src, W_dst, attn):
    raise NotImplementedError("write your pallas kernel here")


# speedup vs baseline: 37.3058x; 37.3058x over previous
import functools

import jax
import jax.numpy as jnp
from jax import lax
from jax.experimental import pallas as pl
from jax.experimental.pallas import tpu as pltpu
from jax.experimental.pallas import tpu_sc as plsc

N = 10000
E = 320000
D = 128
H = 8
P = 16

NC = 2
NS = 16
CHUNK = 80
EDGES_PER_TILE = E // (NC * NS)
NUM_CHUNKS = EDGES_PER_TILE // CHUNK
ROWS_PER_TILE = 624
ROWS_TAIL = N - ROWS_PER_TILE * NS


def _mm_body(x_ref, ws_ref, wd_ref, hs_ref, hd_ref):
    xb = x_ref[...]
    hs_ref[...] = jnp.dot(xb, ws_ref[...], preferred_element_type=jnp.float32)
    hd_ref[...] = jnp.dot(xb, wd_ref[...], preferred_element_type=jnp.float32)


def _project(x, W_src, W_dst):
    blk = 400
    grid = N // blk
    return pl.pallas_call(
        _mm_body,
        grid=(grid,),
        in_specs=[
            pl.BlockSpec((blk, D), lambda i: (i, 0)),
            pl.BlockSpec((D, H * P), lambda i: (0, 0)),
            pl.BlockSpec((D, H * P), lambda i: (0, 0)),
        ],
        out_specs=[
            pl.BlockSpec((blk, H * P), lambda i: (i, 0)),
            pl.BlockSpec((blk, H * P), lambda i: (i, 0)),
        ],
        out_shape=[
            jax.ShapeDtypeStruct((N, H * P), jnp.float32),
            jax.ShapeDtypeStruct((N, H * P), jnp.float32),
        ],
    )(x, W_src, W_dst)


_GATHER_DN = lax.GatherDimensionNumbers(
    offset_dims=(), collapsed_slice_dims=(0,), start_index_map=(0,))


def _shuf(v, idx):
    return lax.gather(v, idx.reshape(16, 1), _GATHER_DN, (1,),
                      mode=lax.GatherScatterMode.PROMISE_IN_BOUNDS)
NPAD = 10240
TILE_ROWS = NPAD // NS
DPAD = NPAD // 8
DTILE = DPAD // NS


def _sc_body(hs_hbm, hd_hbm, src_hbm, dst_hbm, attn_hbm,
             num_hbm, den_hbm,
             acc_num, acc_den, src_v, dst_v, didx_v, hs_buf, hd_buf,
             den_msg, attn_vm, sem1, sem2):
    msg_buf = hs_buf
    cid = lax.axis_index("c")
    sid = lax.axis_index("s")
    zero16 = jnp.zeros((16,), jnp.float32)

    def _zrow(r, _):
        for c in range(H):
            msg_buf[r, pl.ds(16 * c, 16)] = zero16
            den_msg[r, pl.ds(16 * c, 16)] = zero16
        return None
    lax.fori_loop(0, CHUNK, _zrow, None)

    row0 = sid * TILE_ROWS
    for j in range(TILE_ROWS // CHUNK):
        pltpu.sync_copy(msg_buf, acc_num.at[pl.ds(row0 + j * CHUNK, CHUNK)])
    pltpu.sync_copy(den_msg, acc_den.at[pl.ds(sid * DTILE, DTILE)])
    pltpu.sync_copy(attn_hbm, attn_vm)
    plsc.subcore_barrier()

    base0 = cid * (E // NC) + sid * EDGES_PER_TILE
    lane = lax.iota(jnp.int32, 16)

    def _chunk(i, _):
        base = base0 + i * CHUNK
        pltpu.sync_copy(src_hbm.at[pl.ds(base, CHUNK)], src_v)
        pltpu.sync_copy(dst_hbm.at[pl.ds(base, CHUNK)], dst_v)
        g1 = pltpu.async_copy(hs_hbm.at[src_v], hs_buf, sem1)
        g2 = pltpu.async_copy(hd_hbm.at[dst_v], hd_buf, sem2)
        g1.wait()
        g2.wait()
        for g in range(CHUNK // 16):
            didx_v[pl.ds(16 * g, 16)] = jnp.right_shift(
                dst_v[pl.ds(16 * g, 16)], 3)

        def _edge(e, _):
            dv = zero16
            svecs = []
            for h in range(H):
                a = attn_vm[h, :]
                s_ = hs_buf[e, pl.ds(16 * h, 16)]
                d_ = hd_buf[e, pl.ds(16 * h, 16)]
                svecs.append(s_)
                t = s_ + d_
                t = jnp.where(t > 0.0, t, t * jnp.float32(0.2))
                u = t * a
                for dist in (8, 4, 2, 1):
                    u = u + _shuf(u, lane ^ dist)
                dv = jnp.where(lane == h, u, dv)
            wall = jnp.where(lane < H, jnp.exp(dv), 0.0)
            for h in range(H):
                wv = _shuf(wall, jnp.full((16,), h, jnp.int32))
                msg_buf[e, pl.ds(16 * h, 16)] = wv * svecs[h]
            dvec = dst_v[pl.ds((e // 16) * 16, 16)]
            dsp = _shuf(dvec, jnp.full((16,), e % 16, jnp.int32))
            blk_f = jnp.bitwise_and(dsp, 7).astype(jnp.float32)
            one = jnp.float32(1.0)
            for c in range(8):
                d = blk_f - jnp.float32(c)
                m = jnp.maximum(one - d * d, 0.0)
                den_msg[e, pl.ds(16 * c, 16)] = wall * m
            return None
        lax.fori_loop(0, CHUNK, _edge, None)

        pltpu.sync_copy(msg_buf, acc_num.at[dst_v], add=True)
        pltpu.sync_copy(den_msg, acc_den.at[didx_v], add=True)
        return None

    lax.fori_loop(0, NUM_CHUNKS, _chunk, None)
    plsc.subcore_barrier()

    pltpu.sync_copy(acc_num.at[pl.ds(row0, TILE_ROWS)],
                    num_hbm.at[cid, pl.ds(row0, TILE_ROWS)])
    pltpu.sync_copy(acc_den.at[pl.ds(sid * DTILE, DTILE)],
                    den_hbm.at[cid, pl.ds(sid * DTILE, DTILE)])


def _sc_edge(hs, hd, src, dst, attn):
    mesh = plsc.VectorSubcoreMesh(core_axis_name="c", subcore_axis_name="s")
    fn = pl.kernel(
        _sc_body,
        out_type=[
            jax.ShapeDtypeStruct((NC, NPAD, H * P), jnp.float32),
            jax.ShapeDtypeStruct((NC, DPAD, H * P), jnp.float32),
        ],
        mesh=mesh,
        scratch_types=[
            pltpu.VMEM_SHARED((NPAD, H * P), jnp.float32),
            pltpu.VMEM_SHARED((DPAD, H * P), jnp.float32),
            pltpu.VMEM((CHUNK,), jnp.int32),
            pltpu.VMEM((CHUNK,), jnp.int32),
            pltpu.VMEM((CHUNK,), jnp.int32),
            pltpu.VMEM((CHUNK, H * P), jnp.float32),
            pltpu.VMEM((CHUNK, H * P), jnp.float32),
            pltpu.VMEM((CHUNK, H * P), jnp.float32),
            pltpu.VMEM((H, 16), jnp.float32),
            pltpu.SemaphoreType.DMA,
            pltpu.SemaphoreType.DMA,
        ],
    )
    return fn(hs, hd, src, dst, attn)


def _comb_body(num_ref, den_ref, out_ref):
    num = num_ref[0] + num_ref[1]
    den = den_ref[0] + den_ref[1]
    row = lax.broadcasted_iota(jnp.int32, (16, H * P), 0)
    col = lax.broadcasted_iota(jnp.int32, (16, H * P), 1)
    expand = (col // P == row).astype(jnp.float32)
    den_full = jnp.dot(den, expand, preferred_element_type=jnp.float32)
    out_ref[...] = jnp.maximum(num / (den_full + 1e-16), 0.0)


def _combine(num_p, den_flat):
    blk = 400
    grid = N // blk
    return pl.pallas_call(
        _comb_body,
        grid=(grid,),
        in_specs=[
            pl.BlockSpec((NC, blk, H * P), lambda i: (0, i, 0)),
            pl.BlockSpec((NC, blk, 16), lambda i: (0, i, 0)),
        ],
        out_specs=pl.BlockSpec((blk, H * P), lambda i: (i, 0)),
        out_shape=jax.ShapeDtypeStruct((N, H * P), jnp.float32),
    )(num_p, den_flat)


@jax.jit
def kernel(x, edge_index, W_src, W_dst, attn):
    ei = edge_index.astype(jnp.int32)
    src = ei[0]
    dst = ei[1]
    hs, hd = _project(x, W_src, W_dst)
    num_p, den_p = _sc_edge(hs, hd, src, dst, attn)
    den_flat = den_p.reshape(NC, NPAD, 16)
    return _combine(num_p, den_flat)


# speedup vs baseline: 39.9036x; 1.0696x over previous
import functools

import jax
import jax.numpy as jnp
from jax import lax
from jax.experimental import pallas as pl
from jax.experimental.pallas import tpu as pltpu
from jax.experimental.pallas import tpu_sc as plsc

N = 10000
E = 320000
D = 128
H = 8
P = 16

NC = 2
NS = 16
CHUNK = 80
EDGES_PER_TILE = E // (NC * NS)
NUM_CHUNKS = EDGES_PER_TILE // CHUNK
ROWS_PER_TILE = 624
ROWS_TAIL = N - ROWS_PER_TILE * NS


def _mm_body(x_ref, ws_ref, wd_ref, hs_ref, hd_ref):
    xb = x_ref[...]
    hs_ref[...] = jnp.dot(xb, ws_ref[...], preferred_element_type=jnp.float32)
    hd_ref[...] = jnp.dot(xb, wd_ref[...], preferred_element_type=jnp.float32)


def _project(x, W_src, W_dst):
    blk = 400
    grid = N // blk
    return pl.pallas_call(
        _mm_body,
        grid=(grid,),
        in_specs=[
            pl.BlockSpec((blk, D), lambda i: (i, 0)),
            pl.BlockSpec((D, H * P), lambda i: (0, 0)),
            pl.BlockSpec((D, H * P), lambda i: (0, 0)),
        ],
        out_specs=[
            pl.BlockSpec((blk, H * P), lambda i: (i, 0)),
            pl.BlockSpec((blk, H * P), lambda i: (i, 0)),
        ],
        out_shape=[
            jax.ShapeDtypeStruct((N, H * P), jnp.float32),
            jax.ShapeDtypeStruct((N, H * P), jnp.float32),
        ],
    )(x, W_src, W_dst)


_GATHER_DN = lax.GatherDimensionNumbers(
    offset_dims=(), collapsed_slice_dims=(0,), start_index_map=(0,))


def _shuf(v, idx):
    return lax.gather(v, idx.reshape(16, 1), _GATHER_DN, (1,),
                      mode=lax.GatherScatterMode.PROMISE_IN_BOUNDS)
NPAD = 10240
TILE_ROWS = NPAD // NS
DPAD = NPAD // 8
DTILE = DPAD // NS
BQ = 40
NCH = EDGES_PER_TILE // BQ


def _sc_body(hs_hbm, hd_hbm, src_hbm, dst_hbm, attn_hbm,
             num_hbm, den_hbm,
             acc_num, acc_den,
             srcX, dstX, srcY, dstY, didx,
             hsX, hdX, hsY, hdY, dmsg, attn_vm,
             sgx1, sgx2, sgy1, sgy2):
    cid = lax.axis_index("c")
    sid = lax.axis_index("s")
    zero16 = jnp.zeros((16,), jnp.float32)
    lane = lax.iota(jnp.int32, 16)

    def _zrow(r, _):
        for c in range(H):
            hsX[r, pl.ds(16 * c, 16)] = zero16
            dmsg[r, pl.ds(16 * c, 16)] = zero16
        return None
    lax.fori_loop(0, BQ, _zrow, None)

    row0 = sid * TILE_ROWS
    for j in range(TILE_ROWS // BQ):
        pltpu.sync_copy(hsX, acc_num.at[pl.ds(row0 + j * BQ, BQ)])
    for j in range(DTILE // BQ):
        pltpu.sync_copy(dmsg, acc_den.at[pl.ds(sid * DTILE + j * BQ, BQ)])
    pltpu.sync_copy(attn_hbm, attn_vm)
    plsc.subcore_barrier()

    av = [attn_vm[h, :] for h in range(H)]
    base0 = cid * (E // NC) + sid * EDGES_PER_TILE

    def _idx_load(src_v, dst_v, c):
        b = base0 + c * BQ
        pltpu.sync_copy(src_hbm.at[pl.ds(b, BQ)], src_v)
        pltpu.sync_copy(dst_hbm.at[pl.ds(b, BQ)], dst_v)

    def _compute(hs_buf, hd_buf, dst_v):
        for o in (0, 16, 24):
            didx[pl.ds(o, 16)] = jnp.right_shift(dst_v[pl.ds(o, 16)], 3)

        def _edge(e, _):
            dv = zero16
            svecs = []
            for h in range(H):
                s_ = hs_buf[e, pl.ds(16 * h, 16)]
                d_ = hd_buf[e, pl.ds(16 * h, 16)]
                svecs.append(s_)
                t = s_ + d_
                t = jnp.where(t > 0.0, t, t * jnp.float32(0.2))
                u = t * av[h]
                for dist in (8, 4, 2, 1):
                    u = u + _shuf(u, lane ^ dist)
                dv = jnp.where(lane == h, u, dv)
            wall = jnp.where(lane < H, jnp.exp(dv), 0.0)
            for h in range(H):
                wv = _shuf(wall, jnp.full((16,), h, jnp.int32))
                hs_buf[e, pl.ds(16 * h, 16)] = wv * svecs[h]
            off = jnp.minimum((e // 16) * 16, BQ - 16)
            dvec = dst_v[pl.ds(off, 16)]
            dsp = _shuf(dvec, jnp.full((16,), 0, jnp.int32) + (e - off))
            blk_f = jnp.bitwise_and(dsp, 7).astype(jnp.float32)
            one = jnp.float32(1.0)
            for c in range(8):
                d = blk_f - jnp.float32(c)
                m = jnp.maximum(one - d * d, 0.0)
                dmsg[e, pl.ds(16 * c, 16)] = wall * m
            return None
        lax.fori_loop(0, BQ, _edge, None)
        pltpu.sync_copy(hs_buf, acc_num.at[dst_v], add=True)
        pltpu.sync_copy(dmsg, acc_den.at[didx], add=True)

    _idx_load(srcX, dstX, 0)
    pltpu.async_copy(hs_hbm.at[srcX], hsX, sgx1)
    pltpu.async_copy(hd_hbm.at[dstX], hdX, sgx2)

    def _pair(j, _):
        cy = 2 * j + 1
        cn = 2 * j + 2
        cn = jnp.where(cn >= NCH, 0, cn)
        _idx_load(srcY, dstY, cy)
        gy1 = pltpu.async_copy(hs_hbm.at[srcY], hsY, sgy1)
        gy2 = pltpu.async_copy(hd_hbm.at[dstY], hdY, sgy2)
        pltpu.make_async_copy(hs_hbm.at[srcX], hsX, sgx1).wait()
        pltpu.make_async_copy(hd_hbm.at[dstX], hdX, sgx2).wait()
        _compute(hsX, hdX, dstX)
        _idx_load(srcX, dstX, cn)
        pltpu.async_copy(hs_hbm.at[srcX], hsX, sgx1)
        pltpu.async_copy(hd_hbm.at[dstX], hdX, sgx2)
        gy1.wait()
        gy2.wait()
        _compute(hsY, hdY, dstY)
        return None

    lax.fori_loop(0, NCH // 2, _pair, None)
    pltpu.make_async_copy(hs_hbm.at[srcX], hsX, sgx1).wait()
    pltpu.make_async_copy(hd_hbm.at[dstX], hdX, sgx2).wait()
    plsc.subcore_barrier()

    pltpu.sync_copy(acc_num.at[pl.ds(row0, TILE_ROWS)],
                    num_hbm.at[cid, pl.ds(row0, TILE_ROWS)])
    pltpu.sync_copy(acc_den.at[pl.ds(sid * DTILE, DTILE)],
                    den_hbm.at[cid, pl.ds(sid * DTILE, DTILE)])


def _sc_edge(hs, hd, src, dst, attn):
    mesh = plsc.VectorSubcoreMesh(core_axis_name="c", subcore_axis_name="s")
    fn = pl.kernel(
        _sc_body,
        out_type=[
            jax.ShapeDtypeStruct((NC, NPAD, H * P), jnp.float32),
            jax.ShapeDtypeStruct((NC, DPAD, H * P), jnp.float32),
        ],
        mesh=mesh,
        scratch_types=[
            pltpu.VMEM_SHARED((NPAD, H * P), jnp.float32),
            pltpu.VMEM_SHARED((DPAD, H * P), jnp.float32),
            pltpu.VMEM((BQ,), jnp.int32),
            pltpu.VMEM((BQ,), jnp.int32),
            pltpu.VMEM((BQ,), jnp.int32),
            pltpu.VMEM((BQ,), jnp.int32),
            pltpu.VMEM((BQ,), jnp.int32),
            pltpu.VMEM((BQ, H * P), jnp.float32),
            pltpu.VMEM((BQ, H * P), jnp.float32),
            pltpu.VMEM((BQ, H * P), jnp.float32),
            pltpu.VMEM((BQ, H * P), jnp.float32),
            pltpu.VMEM((BQ, H * P), jnp.float32),
            pltpu.VMEM((H, 16), jnp.float32),
            pltpu.SemaphoreType.DMA,
            pltpu.SemaphoreType.DMA,
            pltpu.SemaphoreType.DMA,
            pltpu.SemaphoreType.DMA,
        ],
    )
    return fn(hs, hd, src, dst, attn)


def _comb_body(num_ref, den_ref, out_ref):
    num = num_ref[0] + num_ref[1]
    den = den_ref[0] + den_ref[1]
    row = lax.broadcasted_iota(jnp.int32, (16, H * P), 0)
    col = lax.broadcasted_iota(jnp.int32, (16, H * P), 1)
    expand = (col // P == row).astype(jnp.float32)
    den_full = jnp.dot(den, expand, preferred_element_type=jnp.float32)
    out_ref[...] = jnp.maximum(num / (den_full + 1e-16), 0.0)


def _combine(num_p, den_flat):
    blk = 400
    grid = N // blk
    return pl.pallas_call(
        _comb_body,
        grid=(grid,),
        in_specs=[
            pl.BlockSpec((NC, blk, H * P), lambda i: (0, i, 0)),
            pl.BlockSpec((NC, blk, 16), lambda i: (0, i, 0)),
        ],
        out_specs=pl.BlockSpec((blk, H * P), lambda i: (i, 0)),
        out_shape=jax.ShapeDtypeStruct((N, H * P), jnp.float32),
    )(num_p, den_flat)


@jax.jit
def kernel(x, edge_index, W_src, W_dst, attn):
    ei = edge_index.astype(jnp.int32)
    src = ei[0]
    dst = ei[1]
    hs, hd = _project(x, W_src, W_dst)
    num_p, den_p = _sc_edge(hs, hd, src, dst, attn)
    den_flat = den_p.reshape(NC, NPAD, 16)
    return _combine(num_p, den_flat)


# speedup vs baseline: 41.4022x; 1.0376x over previous
import functools

import jax
import jax.numpy as jnp
from jax import lax
from jax.experimental import pallas as pl
from jax.experimental.pallas import tpu as pltpu
from jax.experimental.pallas import tpu_sc as plsc

N = 10000
E = 320000
D = 128
H = 8
P = 16

NC = 2
NS = 16
CHUNK = 80
EDGES_PER_TILE = E // (NC * NS)
NUM_CHUNKS = EDGES_PER_TILE // CHUNK
ROWS_PER_TILE = 624
ROWS_TAIL = N - ROWS_PER_TILE * NS


def _mm_body(x_ref, ws_ref, wd_ref, hs_ref, hd_ref):
    xb = x_ref[...]
    hs_ref[...] = jnp.dot(xb, ws_ref[...], preferred_element_type=jnp.float32)
    hd_ref[...] = jnp.dot(xb, wd_ref[...], preferred_element_type=jnp.float32)


def _project(x, W_src, W_dst):
    blk = 400
    grid = N // blk
    return pl.pallas_call(
        _mm_body,
        grid=(grid,),
        in_specs=[
            pl.BlockSpec((blk, D), lambda i: (i, 0)),
            pl.BlockSpec((D, H * P), lambda i: (0, 0)),
            pl.BlockSpec((D, H * P), lambda i: (0, 0)),
        ],
        out_specs=[
            pl.BlockSpec((blk, H * P), lambda i: (i, 0)),
            pl.BlockSpec((blk, H * P), lambda i: (i, 0)),
        ],
        out_shape=[
            jax.ShapeDtypeStruct((N, H * P), jnp.float32),
            jax.ShapeDtypeStruct((N, H * P), jnp.float32),
        ],
    )(x, W_src, W_dst)


_GATHER_DN = lax.GatherDimensionNumbers(
    offset_dims=(), collapsed_slice_dims=(0,), start_index_map=(0,))


def _shuf(v, idx):
    return lax.gather(v, idx.reshape(16, 1), _GATHER_DN, (1,),
                      mode=lax.GatherScatterMode.PROMISE_IN_BOUNDS)
NPAD = 10240
TILE_ROWS = NPAD // NS
DPAD = NPAD // 8
DTILE = DPAD // NS
BQ = 40
NCH = EDGES_PER_TILE // BQ


def _sc_body(hs_hbm, hd_hbm, src_hbm, dst_hbm, attn_hbm,
             num_hbm, den_hbm,
             acc_num, acc_den,
             srcX, dstX, srcY, dstY, didx,
             hsX, hdX, hsY, hdY, dmsg, attn_vm,
             sgx1, sgx2, sgy1, sgy2):
    cid = lax.axis_index("c")
    sid = lax.axis_index("s")
    zero16 = jnp.zeros((16,), jnp.float32)
    lane = lax.iota(jnp.int32, 16)

    def _zrow(r, _):
        for c in range(H):
            hsX[r, pl.ds(16 * c, 16)] = zero16
            dmsg[r, pl.ds(16 * c, 16)] = zero16
        return None
    lax.fori_loop(0, BQ, _zrow, None)

    row0 = sid * TILE_ROWS
    for j in range(TILE_ROWS // BQ):
        pltpu.sync_copy(hsX, acc_num.at[pl.ds(row0 + j * BQ, BQ)])
    for j in range(DTILE // BQ):
        pltpu.sync_copy(dmsg, acc_den.at[pl.ds(sid * DTILE + j * BQ, BQ)])
    pltpu.sync_copy(attn_hbm, attn_vm)
    plsc.subcore_barrier()

    av = [attn_vm[h, :] for h in range(H)]
    base0 = cid * (E // NC) + sid * EDGES_PER_TILE

    def _idx_load(src_v, dst_v, c):
        b = base0 + c * BQ
        pltpu.sync_copy(src_hbm.at[pl.ds(b, BQ)], src_v)
        pltpu.sync_copy(dst_hbm.at[pl.ds(b, BQ)], dst_v)

    def _compute(hs_buf, hd_buf, dst_v):
        for o in (0, 16, 24):
            didx[pl.ds(o, 16)] = jnp.right_shift(dst_v[pl.ds(o, 16)], 3)

        def _edge(ep, _):
          for e in (2 * ep, 2 * ep + 1):
            dv = zero16
            svecs = []
            for h in range(H):
                s_ = hs_buf[e, pl.ds(16 * h, 16)]
                d_ = hd_buf[e, pl.ds(16 * h, 16)]
                svecs.append(s_)
                t = s_ + d_
                t = jnp.where(t > 0.0, t, t * jnp.float32(0.2))
                u = t * av[h]
                for dist in (8, 4, 2, 1):
                    u = u + _shuf(u, lane ^ dist)
                dv = jnp.where(lane == h, u, dv)
            wall = jnp.where(lane < H, jnp.exp(dv), 0.0)
            for h in range(H):
                wv = _shuf(wall, jnp.full((16,), h, jnp.int32))
                hs_buf[e, pl.ds(16 * h, 16)] = wv * svecs[h]
            off = jnp.minimum((e // 16) * 16, BQ - 16)
            dvec = dst_v[pl.ds(off, 16)]
            dsp = _shuf(dvec, jnp.full((16,), 0, jnp.int32) + (e - off))
            blk_f = jnp.bitwise_and(dsp, 7).astype(jnp.float32)
            one = jnp.float32(1.0)
            for c in range(8):
                d = blk_f - jnp.float32(c)
                m = jnp.maximum(one - d * d, 0.0)
                dmsg[e, pl.ds(16 * c, 16)] = wall * m
          return None
        lax.fori_loop(0, BQ // 2, _edge, None)
        pltpu.sync_copy(hs_buf, acc_num.at[dst_v], add=True)
        pltpu.sync_copy(dmsg, acc_den.at[didx], add=True)

    _idx_load(srcX, dstX, 0)
    pltpu.async_copy(hs_hbm.at[srcX], hsX, sgx1)
    pltpu.async_copy(hd_hbm.at[dstX], hdX, sgx2)

    def _pair(j, _):
        cy = 2 * j + 1
        cn = 2 * j + 2
        cn = jnp.where(cn >= NCH, 0, cn)
        _idx_load(srcY, dstY, cy)
        gy1 = pltpu.async_copy(hs_hbm.at[srcY], hsY, sgy1)
        gy2 = pltpu.async_copy(hd_hbm.at[dstY], hdY, sgy2)
        pltpu.make_async_copy(hs_hbm.at[srcX], hsX, sgx1).wait()
        pltpu.make_async_copy(hd_hbm.at[dstX], hdX, sgx2).wait()
        _compute(hsX, hdX, dstX)
        _idx_load(srcX, dstX, cn)
        pltpu.async_copy(hs_hbm.at[srcX], hsX, sgx1)
        pltpu.async_copy(hd_hbm.at[dstX], hdX, sgx2)
        gy1.wait()
        gy2.wait()
        _compute(hsY, hdY, dstY)
        return None

    lax.fori_loop(0, NCH // 2, _pair, None)
    pltpu.make_async_copy(hs_hbm.at[srcX], hsX, sgx1).wait()
    pltpu.make_async_copy(hd_hbm.at[dstX], hdX, sgx2).wait()
    plsc.subcore_barrier()

    pltpu.sync_copy(acc_num.at[pl.ds(row0, TILE_ROWS)],
                    num_hbm.at[cid, pl.ds(row0, TILE_ROWS)])
    pltpu.sync_copy(acc_den.at[pl.ds(sid * DTILE, DTILE)],
                    den_hbm.at[cid, pl.ds(sid * DTILE, DTILE)])


def _sc_edge(hs, hd, src, dst, attn):
    mesh = plsc.VectorSubcoreMesh(core_axis_name="c", subcore_axis_name="s")
    fn = pl.kernel(
        _sc_body,
        out_type=[
            jax.ShapeDtypeStruct((NC, NPAD, H * P), jnp.float32),
            jax.ShapeDtypeStruct((NC, DPAD, H * P), jnp.float32),
        ],
        mesh=mesh,
        scratch_types=[
            pltpu.VMEM_SHARED((NPAD, H * P), jnp.float32),
            pltpu.VMEM_SHARED((DPAD, H * P), jnp.float32),
            pltpu.VMEM((BQ,), jnp.int32),
            pltpu.VMEM((BQ,), jnp.int32),
            pltpu.VMEM((BQ,), jnp.int32),
            pltpu.VMEM((BQ,), jnp.int32),
            pltpu.VMEM((BQ,), jnp.int32),
            pltpu.VMEM((BQ, H * P), jnp.float32),
            pltpu.VMEM((BQ, H * P), jnp.float32),
            pltpu.VMEM((BQ, H * P), jnp.float32),
            pltpu.VMEM((BQ, H * P), jnp.float32),
            pltpu.VMEM((BQ, H * P), jnp.float32),
            pltpu.VMEM((H, 16), jnp.float32),
            pltpu.SemaphoreType.DMA,
            pltpu.SemaphoreType.DMA,
            pltpu.SemaphoreType.DMA,
            pltpu.SemaphoreType.DMA,
        ],
    )
    return fn(hs, hd, src, dst, attn)


def _comb_body(num_ref, den_ref, out_ref):
    num = num_ref[0] + num_ref[1]
    den = den_ref[0] + den_ref[1]
    row = lax.broadcasted_iota(jnp.int32, (16, H * P), 0)
    col = lax.broadcasted_iota(jnp.int32, (16, H * P), 1)
    expand = (col // P == row).astype(jnp.float32)
    den_full = jnp.dot(den, expand, preferred_element_type=jnp.float32)
    out_ref[...] = jnp.maximum(num / (den_full + 1e-16), 0.0)


def _combine(num_p, den_flat):
    blk = 400
    grid = N // blk
    return pl.pallas_call(
        _comb_body,
        grid=(grid,),
        in_specs=[
            pl.BlockSpec((NC, blk, H * P), lambda i: (0, i, 0)),
            pl.BlockSpec((NC, blk, 16), lambda i: (0, i, 0)),
        ],
        out_specs=pl.BlockSpec((blk, H * P), lambda i: (i, 0)),
        out_shape=jax.ShapeDtypeStruct((N, H * P), jnp.float32),
    )(num_p, den_flat)


@jax.jit
def kernel(x, edge_index, W_src, W_dst, attn):
    ei = edge_index.astype(jnp.int32)
    src = ei[0]
    dst = ei[1]
    hs, hd = _project(x, W_src, W_dst)
    num_p, den_p = _sc_edge(hs, hd, src, dst, attn)
    den_flat = den_p.reshape(NC, NPAD, 16)
    return _combine(num_p, den_flat)
